# Initial kernel scaffold; baseline (speedup 1.0000x reference)
import jax
import jax.numpy as jnp
from jax.experimental import pallas as pl


def kernel(x, edge_index, edge_weight, W, b):
    raise NotImplementedError("write your pallas kernel here")


# speedup vs baseline: 4.5685x; 4.5685x over previous
import functools

import jax
import jax.numpy as jnp
from jax import lax
from jax.experimental import pallas as pl
from jax.experimental.pallas import tpu as pltpu
from jax.experimental.pallas import tpu_sc as plsc

N = 10000
E = 320000
IN_C = 256
D = 128
DH = 64
HOPS = 3

NC = 2
NS = 16
L = 16

NP = 10240
NROW = NP // NS
CH = 128
T32 = 10112
NCH32 = T32 // CH
EP = 32 * T32
T16 = EP // NS
NCH16 = T16 // CH

_i32 = jnp.int32
_f32 = jnp.float32


def _bcast16(v):
    return lax.broadcast_in_dim(jnp.asarray(v, _i32), (L,), ())


def _rsqrt16(d):
    i = plsc.bitcast(d, _i32)
    y = plsc.bitcast(jnp.asarray(0x5F3759DF, _i32) - lax.shift_right_arithmetic(i, 1), _f32)
    for _ in range(3):
        y = y * (1.5 - 0.5 * d * y * y)
    return jnp.where(d > 0.0, y, 0.0)


def _mm_body(x_ref, w_ref, b_ref, o_ref):
    acc = lax.dot_general(x_ref[...], w_ref[...], (((1,), (1,)), ((), ())),
                          preferred_element_type=_f32)
    o_ref[...] = acc + b_ref[...]


def _matmul(xp, W, b2d):
    BM = 1024
    return pl.pallas_call(
        _mm_body,
        grid=(NP // BM,),
        in_specs=[
            pl.BlockSpec((BM, IN_C), lambda m: (m, 0)),
            pl.BlockSpec((D, IN_C), lambda m: (0, 0)),
            pl.BlockSpec((1, D), lambda m: (0, 0)),
        ],
        out_specs=pl.BlockSpec((BM, D), lambda m: (m, 0)),
        out_shape=jax.ShapeDtypeStruct((NP, D), _f32),
    )(xp, W, b2d)


_MESH = plsc.VectorSubcoreMesh(core_axis_name="c", subcore_axis_name="s")


def _deg_body(col_hbm, ew_hbm, degp0, degp1, colbuf, ewbuf, valbuf, dbuf,
              degout, dacc):
    c = lax.axis_index("c")
    s = lax.axis_index("s")
    tb = (c * NS + s) * T32

    def _zrow(i, _):
        valbuf[i, :] = jnp.zeros((L,), _f32)
        return 0
    lax.fori_loop(0, CH, _zrow, 0)
    for kk in range(NROW // CH):
        pltpu.sync_copy(valbuf, dacc.at[pl.ds(s * NROW + kk * CH, CH)])
    plsc.subcore_barrier()

    def _chunk(ch, _):
        base = tb + ch * CH
        pltpu.sync_copy(col_hbm.at[pl.ds(base, CH)], colbuf)
        pltpu.sync_copy(ew_hbm.at[pl.ds(base, CH)], ewbuf)

        def _fill(j16, _):
            for l in range(L):
                j = j16 * L + l
                v = plsc.load_gather(ewbuf, [_bcast16(j)])
                valbuf[j, :] = v
            return 0
        lax.fori_loop(0, CH // L, _fill, 0)
        pltpu.sync_copy(valbuf, dacc.at[colbuf], add=True)
        return 0
    lax.fori_loop(0, NCH32, _chunk, 0)
    plsc.subcore_barrier()

    pltpu.sync_copy(dacc.at[pl.ds(s * NROW, NROW)], dbuf)
    lanes0 = jnp.zeros((L,), _i32)

    def _extract(v, _):
        rows = v * L + lax.iota(_i32, L)
        degout[pl.ds(v * L, L)] = plsc.load_gather(dbuf, [rows, lanes0])
        return 0
    lax.fori_loop(0, NROW // L, _extract, 0)

    @pl.when(c == 0)
    def _():
        pltpu.sync_copy(degout, degp0.at[pl.ds(s * NROW, NROW)])

    @pl.when(c == 1)
    def _():
        pltpu.sync_copy(degout, degp1.at[pl.ds(s * NROW, NROW)])


_deg_kernel = pl.kernel(
    _deg_body,
    out_type=(jax.ShapeDtypeStruct((NP,), _f32),
              jax.ShapeDtypeStruct((NP,), _f32)),
    mesh=_MESH,
    scratch_types=(
        pltpu.VMEM((CH,), _i32),
        pltpu.VMEM((CH,), _f32),
        pltpu.VMEM((CH, L), _f32),
        pltpu.VMEM((NROW, L), _f32),
        pltpu.VMEM((NROW,), _f32),
        pltpu.VMEM_SHARED((NP, L), _f32),
    ),
    compiler_params=pltpu.CompilerParams(needs_layout_passes=False, use_tc_tiling_on_sc=False),
)


def _w_body(row_hbm, col_hbm, ew_hbm, degp0, degp1, w_hbm, dp0, dp1, rbuf,
            cbuf, ebuf):
    c = lax.axis_index("c")
    s = lax.axis_index("s")
    tb = (c * NS + s) * T32

    pltpu.sync_copy(degp0, dp0)
    pltpu.sync_copy(degp1, dp1)

    def _dis(v, _):
        sl = pl.ds(v * L, L)
        dp0[sl] = _rsqrt16(dp0[sl] + dp1[sl])
        return 0
    lax.fori_loop(0, NP // L, _dis, 0)

    pltpu.sync_copy(row_hbm.at[pl.ds(tb, T32)], rbuf)
    pltpu.sync_copy(col_hbm.at[pl.ds(tb, T32)], cbuf)
    pltpu.sync_copy(ew_hbm.at[pl.ds(tb, T32)], ebuf)

    def _wgrp(g, _):
        sl = pl.ds(g * L, L)
        dr = plsc.load_gather(dp0, [rbuf[sl]])
        dc = plsc.load_gather(dp0, [cbuf[sl]])
        ebuf[sl] = dr * ebuf[sl] * dc
        return 0
    lax.fori_loop(0, T32 // L, _wgrp, 0)
    pltpu.sync_copy(ebuf, w_hbm.at[pl.ds(tb, T32)])


_w_kernel = pl.kernel(
    _w_body,
    out_type=jax.ShapeDtypeStruct((EP,), _f32),
    mesh=_MESH,
    scratch_types=(
        pltpu.VMEM((NP,), _f32),
        pltpu.VMEM((NP,), _f32),
        pltpu.VMEM((T32,), _i32),
        pltpu.VMEM((T32,), _i32),
        pltpu.VMEM((T32,), _f32),
    ),
    compiler_params=pltpu.CompilerParams(needs_layout_passes=False, use_tc_tiling_on_sc=False),
)


def _hop_body(h0, h1, row_hbm, col_hbm, w_hbm, out0, out1, rowsbuf, rixbuf,
              colbuf, wbuf, hacc, sem):
    c = lax.axis_index("c")
    s = lax.axis_index("s")
    tb = s * T16

    def _zrow(i, _):
        for q in range(DH // L):
            rowsbuf[i, pl.ds(q * L, L)] = jnp.zeros((L,), _f32)
        return 0
    lax.fori_loop(0, CH, _zrow, 0)
    for kk in range(NROW // CH):
        pltpu.sync_copy(rowsbuf, hacc.at[pl.ds(s * NROW + kk * CH, CH)])
    pltpu.sync_copy(w_hbm.at[pl.ds(tb, T16)], wbuf)
    plsc.subcore_barrier()

    def _run(hsrc, odst):
        def _chunk(ch, _):
            base = tb + ch * CH
            pltpu.sync_copy(row_hbm.at[pl.ds(base, CH)], rixbuf)
            pltpu.async_copy(hsrc.at[rixbuf], rowsbuf, sem).wait()
            pltpu.sync_copy(col_hbm.at[pl.ds(base, CH)], colbuf)

            def _scale(j16, _):
                for l in range(L):
                    j = j16 * L + l
                    ws = plsc.load_gather(wbuf, [_bcast16(ch * CH + j)])
                    for q in range(DH // L):
                        sl = pl.ds(q * L, L)
                        rowsbuf[j, sl] = rowsbuf[j, sl] * ws
                return 0
            lax.fori_loop(0, CH // L, _scale, 0)
            pltpu.sync_copy(rowsbuf, hacc.at[colbuf], add=True)
            return 0
        lax.fori_loop(0, NCH16, _chunk, 0)
        plsc.subcore_barrier()
        sl = pl.ds(s * NROW, NROW)
        pltpu.sync_copy(hacc.at[sl], odst.at[sl])

    @pl.when(c == 0)
    def _():
        _run(h0, out0)

    @pl.when(c == 1)
    def _():
        _run(h1, out1)


_hop_kernel = pl.kernel(
    _hop_body,
    out_type=(jax.ShapeDtypeStruct((NP, DH), _f32),
              jax.ShapeDtypeStruct((NP, DH), _f32)),
    mesh=_MESH,
    scratch_types=(
        pltpu.VMEM((CH, DH), _f32),
        pltpu.VMEM((CH,), _i32),
        pltpu.VMEM((CH,), _i32),
        pltpu.VMEM((T16,), _f32),
        pltpu.VMEM_SHARED((NP, DH), _f32),
        pltpu.SemaphoreType.DMA,
    ),
    compiler_params=pltpu.CompilerParams(needs_layout_passes=False, use_tc_tiling_on_sc=False),
)


@jax.jit
def kernel(x, edge_index, edge_weight, W, b):
    row = jnp.pad(edge_index[0], (0, EP - E))
    col = jnp.pad(edge_index[1], (0, EP - E))
    ew = jnp.pad(edge_weight, (0, EP - E))
    xp = jnp.pad(x, ((0, NP - N), (0, 0)))

    h = _matmul(xp, W, b.reshape(1, D))
    degp0, degp1 = _deg_kernel(col, ew)
    w = _w_kernel(row, col, ew, degp0, degp1)

    h0 = h[:, :DH]
    h1 = h[:, DH:]
    for _ in range(HOPS):
        h0, h1 = _hop_kernel(h0, h1, row, col, w)
    return jnp.concatenate([h0[:N], h1[:N]], axis=1)


# speedup vs baseline: 8.0919x; 1.7712x over previous
import functools

import jax
import jax.numpy as jnp
from jax import lax
from jax.experimental import pallas as pl
from jax.experimental.pallas import tpu as pltpu
from jax.experimental.pallas import tpu_sc as plsc

N = 10000
E = 320000
IN_C = 256
D = 128
DH = 64
HOPS = 3

NC = 2
NS = 16
L = 16

NP = 10240
NROW = NP // NS
CH = 128
T32 = 10112
NCH32 = T32 // CH
EP = 32 * T32
T16 = EP // NS
NCH16 = T16 // CH

_i32 = jnp.int32
_f32 = jnp.float32


def _bcast16(v):
    return lax.broadcast_in_dim(jnp.asarray(v, _i32), (L,), ())


def _rsqrt16(d):
    i = plsc.bitcast(d, _i32)
    y = plsc.bitcast(jnp.asarray(0x5F3759DF, _i32) - lax.shift_right_arithmetic(i, 1), _f32)
    for _ in range(3):
        y = y * (1.5 - 0.5 * d * y * y)
    return jnp.where(d > 0.0, y, 0.0)


def _mm_body(x_ref, w_ref, b_ref, o_ref):
    acc = lax.dot_general(x_ref[...], w_ref[...], (((1,), (1,)), ((), ())),
                          preferred_element_type=_f32)
    o_ref[...] = acc + b_ref[...]


def _matmul(xp, W, b2d):
    BM = 1024
    return pl.pallas_call(
        _mm_body,
        grid=(NP // BM,),
        in_specs=[
            pl.BlockSpec((BM, IN_C), lambda m: (m, 0)),
            pl.BlockSpec((D, IN_C), lambda m: (0, 0)),
            pl.BlockSpec((1, D), lambda m: (0, 0)),
        ],
        out_specs=pl.BlockSpec((BM, D), lambda m: (m, 0)),
        out_shape=jax.ShapeDtypeStruct((NP, D), _f32),
    )(xp, W, b2d)


_MESH = plsc.VectorSubcoreMesh(core_axis_name="c", subcore_axis_name="s")


def _deg_body(col_hbm, ew_hbm, degp0, degp1, colbuf, ewbuf, valbuf, dbuf,
              degout, dacc):
    c = lax.axis_index("c")
    s = lax.axis_index("s")
    tb = (c * NS + s) * T32

    def _zrow(i, _):
        valbuf[i, :] = jnp.zeros((L,), _f32)
        return 0
    lax.fori_loop(0, CH, _zrow, 0)
    for kk in range(NROW // CH):
        pltpu.sync_copy(valbuf, dacc.at[pl.ds(s * NROW + kk * CH, CH)])
    plsc.subcore_barrier()

    def _chunk(ch, _):
        base = tb + ch * CH
        pltpu.sync_copy(col_hbm.at[pl.ds(base, CH)], colbuf)
        pltpu.sync_copy(ew_hbm.at[pl.ds(base, CH)], ewbuf)

        def _fill(j16, _):
            for l in range(L):
                j = j16 * L + l
                v = plsc.load_gather(ewbuf, [_bcast16(j)])
                valbuf[j, :] = v
            return 0
        lax.fori_loop(0, CH // L, _fill, 0)
        pltpu.sync_copy(valbuf, dacc.at[colbuf], add=True)
        return 0
    lax.fori_loop(0, NCH32, _chunk, 0)
    plsc.subcore_barrier()

    pltpu.sync_copy(dacc.at[pl.ds(s * NROW, NROW)], dbuf)
    lanes0 = jnp.zeros((L,), _i32)

    def _extract(v, _):
        rows = v * L + lax.iota(_i32, L)
        degout[pl.ds(v * L, L)] = plsc.load_gather(dbuf, [rows, lanes0])
        return 0
    lax.fori_loop(0, NROW // L, _extract, 0)

    @pl.when(c == 0)
    def _():
        pltpu.sync_copy(degout, degp0.at[pl.ds(s * NROW, NROW)])

    @pl.when(c == 1)
    def _():
        pltpu.sync_copy(degout, degp1.at[pl.ds(s * NROW, NROW)])


_deg_kernel = pl.kernel(
    _deg_body,
    out_type=(jax.ShapeDtypeStruct((NP,), _f32),
              jax.ShapeDtypeStruct((NP,), _f32)),
    mesh=_MESH,
    scratch_types=(
        pltpu.VMEM((CH,), _i32),
        pltpu.VMEM((CH,), _f32),
        pltpu.VMEM((CH, L), _f32),
        pltpu.VMEM((NROW, L), _f32),
        pltpu.VMEM((NROW,), _f32),
        pltpu.VMEM_SHARED((NP, L), _f32),
    ),
    compiler_params=pltpu.CompilerParams(needs_layout_passes=False, use_tc_tiling_on_sc=False),
)


def _w_body(row_hbm, col_hbm, ew_hbm, degp0, degp1, w_hbm, dp0, dp1, rbuf,
            cbuf, ebuf):
    c = lax.axis_index("c")
    s = lax.axis_index("s")
    tb = (c * NS + s) * T32

    pltpu.sync_copy(degp0, dp0)
    pltpu.sync_copy(degp1, dp1)

    def _dis(v, _):
        sl = pl.ds(v * L, L)
        dp0[sl] = _rsqrt16(dp0[sl] + dp1[sl])
        return 0
    lax.fori_loop(0, NP // L, _dis, 0)

    pltpu.sync_copy(row_hbm.at[pl.ds(tb, T32)], rbuf)
    pltpu.sync_copy(col_hbm.at[pl.ds(tb, T32)], cbuf)
    pltpu.sync_copy(ew_hbm.at[pl.ds(tb, T32)], ebuf)

    def _wgrp(g, _):
        sl = pl.ds(g * L, L)
        dr = plsc.load_gather(dp0, [rbuf[sl]])
        dc = plsc.load_gather(dp0, [cbuf[sl]])
        ebuf[sl] = dr * ebuf[sl] * dc
        return 0
    lax.fori_loop(0, T32 // L, _wgrp, 0)
    pltpu.sync_copy(ebuf, w_hbm.at[pl.ds(tb, T32)])


_w_kernel = pl.kernel(
    _w_body,
    out_type=jax.ShapeDtypeStruct((EP,), _f32),
    mesh=_MESH,
    scratch_types=(
        pltpu.VMEM((NP,), _f32),
        pltpu.VMEM((NP,), _f32),
        pltpu.VMEM((T32,), _i32),
        pltpu.VMEM((T32,), _i32),
        pltpu.VMEM((T32,), _f32),
    ),
    compiler_params=pltpu.CompilerParams(needs_layout_passes=False, use_tc_tiling_on_sc=False),
)


def _hop_body(h0, h1, row2_hbm, col2_hbm, w_hbm, out0, out1, rowsA, rowsB,
              rix2d, col2d, wbuf, hacc, semA, semB):
    c = lax.axis_index("c")
    s = lax.axis_index("s")

    pltpu.sync_copy(row2_hbm.at[pl.ds(s * NCH16, NCH16)], rix2d)
    pltpu.sync_copy(col2_hbm.at[pl.ds(s * NCH16, NCH16)], col2d)
    pltpu.sync_copy(w_hbm.at[pl.ds(s * T16, T16)], wbuf)

    def _zrow(i, _):
        for q in range(DH // L):
            rowsA[i, pl.ds(q * L, L)] = jnp.zeros((L,), _f32)
        return 0
    lax.fori_loop(0, CH, _zrow, 0)
    for kk in range(NROW // CH):
        pltpu.sync_copy(rowsA, hacc.at[pl.ds(s * NROW + kk * CH, CH)])
    plsc.subcore_barrier()

    def _run(hsrc, odst):
        def _gather(ch, buf, sem):
            pltpu.async_copy(hsrc.at[rix2d.at[ch]], buf, sem)

        def _wait(ch, buf, sem):
            pltpu.make_async_copy(hsrc.at[rix2d.at[ch]], buf, sem).wait()

        def _scale(ch, buf):
            def _sc(j16, _):
                for l in range(L):
                    j = j16 * L + l
                    ws = plsc.load_gather(wbuf, [_bcast16(ch * CH + j)])
                    for q in range(DH // L):
                        sl = pl.ds(q * L, L)
                        buf[j, sl] = buf[j, sl] * ws
                return 0
            lax.fori_loop(0, CH // L, _sc, 0)

        def _scatter(ch, buf):
            pltpu.sync_copy(buf, hacc.at[col2d.at[ch]], add=True)

        _gather(0, rowsA, semA)

        def _pair(i, _):
            chA = 2 * i
            chB = 2 * i + 1
            _wait(chA, rowsA, semA)
            _gather(chB, rowsB, semB)
            _scale(chA, rowsA)
            _scatter(chA, rowsA)
            _wait(chB, rowsB, semB)

            @pl.when(i < NCH16 // 2 - 1)
            def _():
                _gather(chB + 1, rowsA, semA)

            _scale(chB, rowsB)
            _scatter(chB, rowsB)
            return 0
        lax.fori_loop(0, NCH16 // 2, _pair, 0)
        plsc.subcore_barrier()
        sl = pl.ds(s * NROW, NROW)
        pltpu.sync_copy(hacc.at[sl], odst.at[sl])

    @pl.when(c == 0)
    def _():
        _run(h0, out0)

    @pl.when(c == 1)
    def _():
        _run(h1, out1)


_hop_kernel = pl.kernel(
    _hop_body,
    out_type=(jax.ShapeDtypeStruct((NP, DH), _f32),
              jax.ShapeDtypeStruct((NP, DH), _f32)),
    mesh=_MESH,
    scratch_types=(
        pltpu.VMEM((CH, DH), _f32),
        pltpu.VMEM((CH, DH), _f32),
        pltpu.VMEM((NCH16, CH), _i32),
        pltpu.VMEM((NCH16, CH), _i32),
        pltpu.VMEM((T16,), _f32),
        pltpu.VMEM_SHARED((NP, DH), _f32),
        pltpu.SemaphoreType.DMA,
        pltpu.SemaphoreType.DMA,
    ),
    compiler_params=pltpu.CompilerParams(needs_layout_passes=False, use_tc_tiling_on_sc=False),
)


@jax.jit
def kernel(x, edge_index, edge_weight, W, b):
    row = jnp.pad(edge_index[0], (0, EP - E))
    col = jnp.pad(edge_index[1], (0, EP - E))
    ew = jnp.pad(edge_weight, (0, EP - E))
    xp = jnp.pad(x, ((0, NP - N), (0, 0)))

    h = _matmul(xp, W, b.reshape(1, D))
    degp0, degp1 = _deg_kernel(col, ew)
    w = _w_kernel(row, col, ew, degp0, degp1)

    row2 = row.reshape(EP // CH, CH)
    col2 = col.reshape(EP // CH, CH)
    h0 = h[:, :DH]
    h1 = h[:, DH:]
    for _ in range(HOPS):
        h0, h1 = _hop_kernel(h0, h1, row2, col2, w)
    return jnp.concatenate([h0[:N], h1[:N]], axis=1)
